# baseline (speedup 1.0000x reference)
import jax
import jax.numpy as jnp
from jax import lax
from jax.experimental import pallas as pl
from jax.experimental.pallas import tpu as pltpu
from jax.experimental.pallas import tpu_sc as plsc

NC = 2
NS = 16
NW = NC * NS

CHUNK = 16
NBUF = 2


def _make_sc_call(B, S, L, D, V):
    SENT = B * S
    assert SENT % NW == 0
    sent_per_w = SENT // NW
    ipc = CHUNK * L
    bounds = list(range(0, ipc, 128)) + [ipc]
    segs = [(o, n - o) for o, n in zip(bounds[:-1], bounds[1:])]
    assert sent_per_w % (CHUNK * 4) == 0
    n_chunks = sent_per_w // CHUNK

    mesh = plsc.VectorSubcoreMesh(core_axis_name="c", subcore_axis_name="s")

    @pl.kernel(
        out_type=jax.ShapeDtypeStruct((SENT, D), jnp.float32),
        mesh=mesh,
        compiler_params=pltpu.CompilerParams(use_tc_tiling_on_sc=True,
                                             needs_layout_passes=False),
        scratch_types=[
            pltpu.VMEM((4 * ipc,), jnp.int32),
            pltpu.VMEM((4 * ipc * 16,), jnp.int32),
            pltpu.VMEM((NBUF * ipc, 2 * D), jnp.float32),
            pltpu.VMEM((CHUNK, D), jnp.float32),
            pltpu.VMEM((L, D), jnp.float32),
            pltpu.SemaphoreType.DMA,
            pltpu.SemaphoreType.DMA,
            pltpu.SemaphoreType.DMA,
            pltpu.SemaphoreType.DMA,
        ],
    )
    def sc_call(tpair_hbm, pid_hbm, off_hbm, mask_hbm, out_hbm,
                idx_v, off_v, rows_v, out_v, mask_v,
                sg0, sg1, si0, si1):
        wid = lax.axis_index("s") * NC + lax.axis_index("c")
        pltpu.sync_copy(mask_hbm, mask_v)
        sent_base = wid * sent_per_w
        idx_base = sent_base * L
        sems_g = [sg0, sg1]
        sems_i = [si0, si1]

        def stage_idx(c, islot, sem):
            src = pl.ds(idx_base + c * ipc, ipc)
            osrc = pl.ds((idx_base + c * ipc) * 16, ipc * 16)
            pltpu.async_copy(pid_hbm.at[src],
                             idx_v.at[pl.ds(islot * ipc, ipc)], sem)
            pltpu.async_copy(off_hbm.at[osrc],
                             off_v.at[pl.ds(islot * ipc * 16, ipc * 16)],
                             sem)

        def wait_idx(c, islot, sem):
            src = pl.ds(idx_base + c * ipc, ipc)
            osrc = pl.ds((idx_base + c * ipc) * 16, ipc * 16)
            pltpu.make_async_copy(
                pid_hbm.at[src],
                idx_v.at[pl.ds(islot * ipc, ipc)], sem).wait()
            pltpu.make_async_copy(
                off_hbm.at[osrc],
                off_v.at[pl.ds(islot * ipc * 16, ipc * 16)], sem).wait()

        def fire_gathers(islot, b):
            for o, n in segs:
                pltpu.async_copy(
                    tpair_hbm.at[idx_v.at[pl.ds(islot * ipc + o, n)]],
                    rows_v.at[pl.ds(b * ipc + o, n)], sems_g[b])

        def drain_gathers(islot, b):
            for o, n in segs:
                pltpu.make_async_copy(
                    tpair_hbm.at[idx_v.at[pl.ds(islot * ipc + o, n)]],
                    rows_v.at[pl.ds(b * ipc + o, n)], sems_g[b]).wait()

        def compute(c, islot, b):
            masks = [[mask_v[l, pl.ds(dc * 16, 16)] for l in range(L)]
                     for dc in range(D // 16)]

            @pl.loop(0, CHUNK)
            def _sent(s):
                base = s * L
                accs = [None] * (D // 16)
                for l in range(L):
                    col0 = off_v[pl.ds((islot * ipc + base + l) * 16, 16)]
                    row16 = jnp.broadcast_to(b * ipc + base + l, (16,))
                    for dc in range(D // 16):
                        v = plsc.load_gather(rows_v,
                                             [row16, col0 + (dc * 16)])
                        t = v * masks[dc][l]
                        accs[dc] = t if accs[dc] is None else accs[dc] + t
                for dc in range(D // 16):
                    out_v[s, pl.ds(dc * 16, 16)] = accs[dc]

            pltpu.sync_copy(out_v,
                            out_hbm.at[pl.ds(sent_base + c * CHUNK, CHUNK)])

        for b in range(NBUF):
            stage_idx(b, b, sems_i[b])
            wait_idx(b, b, sems_i[b])
            fire_gathers(b, b)

        @pl.loop(0, n_chunks, step=4)
        def _chunks(c):
            for k in range(4):
                cc = c + k
                b = k % NBUF
                islot = k
                nslot = (k + 2) % 4
                nxt = cc + NBUF
                drain_gathers(islot, b)

                @pl.when(nxt < n_chunks)
                def _prefetch_idx():
                    stage_idx(nxt, nslot, sems_i[b])

                compute(cc, islot, b)

                @pl.when(nxt < n_chunks)
                def _fire_next():
                    wait_idx(nxt, nslot, sems_i[b])
                    fire_gathers(nslot, b)

    return sc_call


def kernel(stories, table, mask):
    B, S, L = stories.shape
    V, D = table.shape
    idx_flat = stories.astype(jnp.int32).reshape(-1)
    pair_id = idx_flat >> 1
    col0 = (((idx_flat & 1) * D)[:, None]
            + jnp.arange(16, dtype=jnp.int32)).reshape(-1)
    one = jax.lax.optimization_barrier(jnp.ones((), jnp.float32))
    t_pair = table.reshape(V // 2, 2 * D) * one
    sc_call = _make_sc_call(B, S, L, D, V)
    out = sc_call(t_pair, pair_id, col0, mask.astype(jnp.float32))
    return out.reshape(B, S, D)


# speedup vs baseline: 2.0431x; 2.0431x over previous
import jax
import jax.numpy as jnp
from jax import lax
from jax.experimental import pallas as pl
from jax.experimental.pallas import tpu as pltpu
from jax.experimental.pallas import tpu_sc as plsc

NC = 2
NS = 16
NW = NC * NS

IDX_PER_STREAM = 128
NBUF = 2


def _gcd(a, b):
    while b:
        a, b = b, a % b
    return a


def _make_sc_call(B, S, L, D, V):
    SENT = B * S
    assert SENT % NW == 0
    sent_per_w = SENT // NW
    chunk = IDX_PER_STREAM // _gcd(IDX_PER_STREAM, L)
    ipc = chunk * L
    n_streams = ipc // IDX_PER_STREAM
    assert sent_per_w % (chunk * NBUF) == 0
    n_chunks = sent_per_w // chunk

    mesh = plsc.VectorSubcoreMesh(core_axis_name="c", subcore_axis_name="s")

    @pl.kernel(
        out_type=jax.ShapeDtypeStruct((SENT, D), jnp.float32),
        mesh=mesh,
        compiler_params=pltpu.CompilerParams(use_tc_tiling_on_sc=False),
        scratch_types=[
            pltpu.VMEM((NBUF, ipc), jnp.int32),
            pltpu.VMEM((NBUF, ipc, D), jnp.float32),
            pltpu.VMEM((NBUF, chunk, D), jnp.float32),
            pltpu.VMEM((L, D), jnp.float32),
            pltpu.SemaphoreType.DMA,
            pltpu.SemaphoreType.DMA,
            pltpu.SemaphoreType.DMA,
            pltpu.SemaphoreType.DMA,
            pltpu.SemaphoreType.DMA,
            pltpu.SemaphoreType.DMA,
        ],
    )
    def sc_call(table_hbm, idx_hbm, mask_hbm, out_hbm,
                idx_v, rows_v, out_v, mask_v, sg0, sg1, si0, si1, so0, so1):
        wid = lax.axis_index("s") * NC + lax.axis_index("c")
        pltpu.sync_copy(mask_hbm, mask_v)
        sent_base = wid * sent_per_w
        idx_base = sent_base * L
        sems_g = [sg0, sg1]
        sems_i = [si0, si1]
        sems_o = [so0, so1]

        def idx_src(c):
            return idx_hbm.at[pl.ds(idx_base + c * ipc, ipc)]

        def out_dst(c):
            return out_hbm.at[pl.ds(sent_base + c * chunk, chunk)]

        def fire_gathers(b):
            for j in range(n_streams):
                js = pl.ds(j * IDX_PER_STREAM, IDX_PER_STREAM)
                pltpu.async_copy(table_hbm.at[idx_v.at[b, js]],
                                 rows_v.at[b, js], sems_g[b])

        def drain_gathers(b):
            for j in range(n_streams):
                js = pl.ds(j * IDX_PER_STREAM, IDX_PER_STREAM)
                pltpu.make_async_copy(table_hbm.at[idx_v.at[b, js]],
                                      rows_v.at[b, js], sems_g[b]).wait()

        def compute(c, b):
            for dc in range(D // 16):
                dsl = pl.ds(dc * 16, 16)
                m = [mask_v[l, dsl] for l in range(L)]

                @pl.loop(0, chunk, unroll=2)
                def _sent(s):
                    base = s * L
                    acc = rows_v[b, base, dsl] * m[0]
                    for l in range(1, L):
                        acc = acc + rows_v[b, base + l, dsl] * m[l]
                    out_v[b, s, dsl] = acc

            pltpu.async_copy(out_v.at[b], out_dst(c), sems_o[b])

        for b in range(NBUF):
            pltpu.sync_copy(idx_src(b), idx_v.at[b])
            fire_gathers(b)

        @pl.loop(0, n_chunks, step=NBUF)
        def _chunks(c):
            for b in range(NBUF):
                cc = c + b
                nxt = cc + NBUF
                drain_gathers(b)

                @pl.when(nxt < n_chunks)
                def _prefetch_idx():
                    pltpu.async_copy(idx_src(nxt), idx_v.at[b], sems_i[b])

                @pl.when(cc >= NBUF)
                def _drain_out():
                    pltpu.make_async_copy(out_v.at[b], out_dst(cc - NBUF),
                                          sems_o[b]).wait()

                compute(cc, b)

                @pl.when(nxt < n_chunks)
                def _fire_next():
                    pltpu.make_async_copy(idx_src(nxt), idx_v.at[b],
                                          sems_i[b]).wait()
                    fire_gathers(b)

        for b in range(NBUF):
            pltpu.make_async_copy(out_v.at[b], out_dst(n_chunks - NBUF + b),
                                  sems_o[b]).wait()

    return sc_call


def kernel(stories, table, mask):
    B, S, L = stories.shape
    V, D = table.shape
    idx_flat = stories.astype(jnp.int32).reshape(-1)
    sc_call = _make_sc_call(B, S, L, D, V)
    out = sc_call(table, idx_flat, mask.astype(jnp.float32))
    return out.reshape(B, S, D)
